# baseline (speedup 1.0000x reference)
import functools

import jax
import jax.numpy as jnp
from jax import lax
from jax.experimental import pallas as pl
from jax.experimental.pallas import tpu as pltpu
from jax.experimental.pallas import tpu_sc as plsc

_R = 2
_KS = 2 * _R + 1
_K2 = _KS * _KS
_CTR = _K2 // 2
_TOPK = 10
_TAU = 0.1
_EPS = 1e-8
_TH = 16
_SLAB = _TH + 8
_NW = 32


def _dma_prologue(xp_hbm, slab, sem, th):
    b = pl.program_id(0)
    t = pl.program_id(1)
    nt = pl.num_programs(1)
    step = b * nt + t
    cur = jax.lax.rem(step, 2)
    nxt = 1 - cur

    def band_copy(bb, tt, buf, s):
        return pltpu.make_async_copy(
            xp_hbm.at[bb, :, pl.ds(tt * th, _SLAB), :], slab.at[buf],
            sem.at[s])

    @pl.when(step == 0)
    def _():
        band_copy(0, 0, 0, 0).start()

    band_copy(b, t, cur, cur).wait()

    @pl.when(step < pl.num_programs(0) * nt - 1)
    def _():
        nb = jnp.where(t == nt - 1, b + 1, b)
        ntt = jnp.where(t == nt - 1, 0, t + 1)
        band_copy(nb, ntt, nxt, nxt).start()

    return slab.at[cur]


def _build_shifted(cs, sh, C, W, th, with_norms):
    f32 = jnp.float32
    sn2f = jnp.zeros((_SLAB, W + 2 * _R), f32) if with_norms else None
    for c in range(C):
        v = cs[c]
        if with_norms:
            sn2f = sn2f + v * v
        for j in range(_KS):
            vj = v[:, j:j + W]
            for i in range(_KS):
                sh[i * _KS + j, c] = vj[i:i + th, :]
    return sn2f


def _sims_body(xp_hbm, o_ref, slab, sh, sem):
    th = _TH
    C = slab.shape[1]
    W = o_ref.shape[3]
    f32 = jnp.float32
    cs = _dma_prologue(xp_hbm, slab, sem, th)
    sn2f = _build_shifted(cs, sh, C, W, th, True)
    inv = jax.lax.rsqrt(jnp.maximum(sn2f, f32(_EPS * _EPS)))
    inv_c = inv[_R:_R + th, _R:_R + W]
    for i in range(_KS):
        js = [j for j in range(_KS) if i * _KS + j != _CTR]
        accs = {j: jnp.zeros((th, W), f32) for j in js}
        for c in range(C):
            ctr = sh[_CTR, c]
            for j in js:
                accs[j] = accs[j] + sh[i * _KS + j, c] * ctr
        for j in js:
            o_ref[0, i * _KS + j] = accs[j] * (inv[i:i + th, j:j + W]
                                               * inv_c) * f32(1.0 / _TAU)
    o_ref[0, _CTR] = jnp.full((th, W), 1.0 / _TAU, f32)


def _topk_sc_body(ppw, nchunk, sims_hbm, out_hbm, buf, sem):
    f32 = jnp.float32
    wid = lax.axis_index("s") * 2 + lax.axis_index("c")
    b = jax.lax.rem(wid, 2)
    start = (wid // 2) * ppw

    @pl.when(wid < 2 * nchunk)
    def _():
        cp = pltpu.make_async_copy(
            sims_hbm.at[b, :, pl.ds(start, ppw)], buf, sem)
        cp.start()
        cp.wait()

        def grp(g, carry):
            off = pl.multiple_of(g * 16, 16)
            s = [buf[o, pl.ds(off, 16)] for o in range(_K2)]
            mx = s[0]
            for o in range(1, _K2):
                mx = jnp.maximum(mx, s[o])
            neg = jnp.full((16,), -jnp.inf, f32)
            thr = mx
            for _ in range(_TOPK - 1):
                nm = neg
                for o in range(_K2):
                    nm = jnp.maximum(nm, jnp.where(s[o] < thr, s[o], neg))
                thr = nm
            den = jnp.zeros((16,), f32)
            es = []
            for o in range(_K2):
                e = jnp.where(s[o] >= thr, jnp.exp(s[o] - mx), f32(0.0))
                es.append(e)
                den = den + e
            r = f32(1.0) / den
            for o in range(_K2):
                buf[o, pl.ds(off, 16)] = es[o] * r
            return carry

        lax.fori_loop(0, ppw // 16, grp, 0)
        cp2 = pltpu.make_async_copy(
            buf, out_hbm.at[b, :, pl.ds(start, ppw)], sem)
        cp2.start()
        cp2.wait()


def _mix_body(gamma_ref, xp_hbm, w_ref, o_ref, slab, sh, sem):
    th = _TH
    C = slab.shape[1]
    W = o_ref.shape[3]
    f32 = jnp.float32
    cs = _dma_prologue(xp_hbm, slab, sem, th)
    _build_shifted(cs, sh, C, W, th, False)
    gamma = gamma_ref[0]
    for i in range(_KS):
        ws = [w_ref[0, i * _KS + j] for j in range(_KS)]
        for c in range(C):
            acc = o_ref[0, c] if i > 0 else jnp.zeros((th, W), f32)
            for j in range(_KS):
                acc = acc + ws[j] * sh[i * _KS + j, c]
            o_ref[0, c] = acc
    for c in range(C):
        o_ref[0, c] = sh[_CTR, c] + gamma * o_ref[0, c]


def kernel(x, gamma):
    B, C, H, W = x.shape
    HW = H * W
    ppw = 3584
    nchunk = HW // ppw
    xp = jnp.pad(x, ((0, 0), (0, 0), (_R, _SLAB - _TH - _R), (_R, _R)))
    g = jnp.reshape(gamma, (1,)).astype(x.dtype)

    sims = pl.pallas_call(
        _sims_body,
        grid=(B, H // _TH),
        in_specs=[pl.BlockSpec(memory_space=pl.ANY)],
        out_specs=pl.BlockSpec((1, _K2, _TH, W), lambda b, t: (b, 0, t, 0)),
        out_shape=jax.ShapeDtypeStruct((B, _K2, H, W), jnp.float32),
        scratch_shapes=[
            pltpu.VMEM((2, C, _SLAB, W + 2 * _R), jnp.float32),
            pltpu.VMEM((_K2, C, _TH, W), jnp.float32),
            pltpu.SemaphoreType.DMA((2,)),
        ],
    )(xp)

    mesh = plsc.VectorSubcoreMesh(core_axis_name="c", subcore_axis_name="s")
    weights = pl.kernel(
        functools.partial(_topk_sc_body, ppw, nchunk),
        mesh=mesh,
        out_type=jax.ShapeDtypeStruct((B, _K2, HW), jnp.float32),
        scratch_types=[
            pltpu.VMEM((_K2, ppw), jnp.float32),
            pltpu.SemaphoreType.DMA,
        ],
    )(sims.reshape(B, _K2, HW))

    out = pl.pallas_call(
        _mix_body,
        grid=(B, H // _TH),
        in_specs=[
            pl.BlockSpec(memory_space=pltpu.SMEM),
            pl.BlockSpec(memory_space=pl.ANY),
            pl.BlockSpec((1, _K2, _TH, W), lambda b, t: (b, 0, t, 0)),
        ],
        out_specs=pl.BlockSpec((1, C, _TH, W), lambda b, t: (b, 0, t, 0)),
        out_shape=jax.ShapeDtypeStruct((B, C, H, W), x.dtype),
        scratch_shapes=[
            pltpu.VMEM((2, C, _SLAB, W + 2 * _R), jnp.float32),
            pltpu.VMEM((_K2, C, _TH, W), jnp.float32),
            pltpu.SemaphoreType.DMA((2,)),
        ],
    )(g, xp, weights.reshape(B, _K2, H, W))
    return out


# speedup vs baseline: 1.1063x; 1.1063x over previous
import functools

import jax
import jax.numpy as jnp
from jax import lax
from jax.experimental import pallas as pl
from jax.experimental.pallas import tpu as pltpu
from jax.experimental.pallas import tpu_sc as plsc

_R = 2
_KS = 2 * _R + 1
_K2 = _KS * _KS
_CTR = _K2 // 2
_TOPK = 10
_TAU = 0.1
_EPS = 1e-8
_TH = 16
_SLAB = _TH + 8
_NW = 32


def _dma_prologue(xp_hbm, slab, sem, th):
    b = pl.program_id(0)
    t = pl.program_id(1)
    nt = pl.num_programs(1)
    step = b * nt + t
    cur = jax.lax.rem(step, 2)
    nxt = 1 - cur

    def band_copy(bb, tt, buf, s):
        return pltpu.make_async_copy(
            xp_hbm.at[bb, :, pl.ds(tt * th, _SLAB), :], slab.at[buf],
            sem.at[s])

    @pl.when(step == 0)
    def _():
        band_copy(0, 0, 0, 0).start()

    band_copy(b, t, cur, cur).wait()

    @pl.when(step < pl.num_programs(0) * nt - 1)
    def _():
        nb = jnp.where(t == nt - 1, b + 1, b)
        ntt = jnp.where(t == nt - 1, 0, t + 1)
        band_copy(nb, ntt, nxt, nxt).start()

    return slab.at[cur]


def _build_shifted(cs, sh, C, W, th, with_norms):
    f32 = jnp.float32
    sn2f = jnp.zeros((_SLAB, W + 2 * _R), f32) if with_norms else None
    for c in range(C):
        v = cs[c]
        if with_norms:
            sn2f = sn2f + v * v
        for j in range(_KS):
            vj = v[:, j:j + W]
            for i in range(_KS):
                sh[i * _KS + j, c] = vj[i:i + th, :]
    return sn2f


def _sims_body(xp_hbm, o_ref, slab, sh, sem):
    th = _TH
    C = slab.shape[1]
    W = o_ref.shape[3]
    f32 = jnp.float32
    cs = _dma_prologue(xp_hbm, slab, sem, th)
    sn2f = _build_shifted(cs, sh, C, W, th, True)
    inv = jax.lax.rsqrt(jnp.maximum(sn2f, f32(_EPS * _EPS)))
    inv_c = inv[_R:_R + th, _R:_R + W]
    for i in range(_KS):
        js = [j for j in range(_KS) if i * _KS + j != _CTR]
        accs = {j: jnp.zeros((th, W), f32) for j in js}
        for c in range(C):
            ctr = sh[_CTR, c]
            for j in js:
                accs[j] = accs[j] + sh[i * _KS + j, c] * ctr
        for j in js:
            o_ref[0, i * _KS + j] = accs[j] * (inv[i:i + th, j:j + W]
                                               * inv_c) * f32(1.0 / _TAU)
    o_ref[0, _CTR] = jnp.full((th, W), 1.0 / _TAU, f32)


def _topk_sc_body(ppw, nchunk, sims_hbm, out_hbm, buf, sem):
    f32 = jnp.float32
    wid = lax.axis_index("s") * 2 + lax.axis_index("c")
    b = jax.lax.rem(wid, 2)
    start = (wid // 2) * ppw

    @pl.when(wid < 2 * nchunk)
    def _():
        cp = pltpu.make_async_copy(
            sims_hbm.at[b, :, pl.ds(start, ppw)], buf, sem)
        cp.start()
        cp.wait()

        def grp(g, carry):
            off = pl.multiple_of(g * 16, 16)
            s = [buf[o, pl.ds(off, 16)] for o in range(_K2)]
            mx = s[0]
            for o in range(1, _K2):
                mx = jnp.maximum(mx, s[o])
            neg = jnp.full((16,), -jnp.inf, f32)
            thr = mx
            for _ in range(_TOPK - 1):
                nm = neg
                for o in range(_K2):
                    nm = jnp.maximum(nm, jnp.where(s[o] < thr, s[o], neg))
                thr = nm
            den = jnp.zeros((16,), f32)
            es = []
            for o in range(_K2):
                e = jnp.where(s[o] >= thr, jnp.exp(s[o] - mx), f32(0.0))
                es.append(e)
                den = den + e
            r = f32(1.0) / den
            for o in range(_K2):
                buf[o, pl.ds(off, 16)] = es[o] * r
            return carry

        lax.fori_loop(0, ppw // 16, grp, 0)
        cp2 = pltpu.make_async_copy(
            buf, out_hbm.at[b, :, pl.ds(start, ppw)], sem)
        cp2.start()
        cp2.wait()


def _mix_body(gamma_ref, xp_hbm, w_ref, o_ref, slab, shj, sem):
    th = _TH
    C = slab.shape[1]
    W = o_ref.shape[3]
    f32 = jnp.float32
    cs = _dma_prologue(xp_hbm, slab, sem, th)
    for c in range(C):
        v = cs[c]
        for j in range(_KS):
            shj[j, c] = v[:, j:j + W]
    gamma = gamma_ref[0]
    for i in range(_KS):
        ws = [w_ref[0, i * _KS + j] for j in range(_KS)]
        for c in range(C):
            acc = o_ref[0, c] if i > 0 else jnp.zeros((th, W), f32)
            for j in range(_KS):
                acc = acc + ws[j] * shj[j, c, i:i + th, :]
            o_ref[0, c] = acc
    for c in range(C):
        o_ref[0, c] = shj[_R, c, _R:_R + th, :] + gamma * o_ref[0, c]


def kernel(x, gamma):
    B, C, H, W = x.shape
    HW = H * W
    ppw = 3584
    nchunk = HW // ppw
    xp = jnp.pad(x, ((0, 0), (0, 0), (_R, _SLAB - _TH - _R), (_R, _R)))
    g = jnp.reshape(gamma, (1,)).astype(x.dtype)

    sims = pl.pallas_call(
        _sims_body,
        grid=(B, H // _TH),
        in_specs=[pl.BlockSpec(memory_space=pl.ANY)],
        out_specs=pl.BlockSpec((1, _K2, _TH, W), lambda b, t: (b, 0, t, 0)),
        out_shape=jax.ShapeDtypeStruct((B, _K2, H, W), jnp.float32),
        scratch_shapes=[
            pltpu.VMEM((2, C, _SLAB, W + 2 * _R), jnp.float32),
            pltpu.VMEM((_K2, C, _TH, W), jnp.float32),
            pltpu.SemaphoreType.DMA((2,)),
        ],
    )(xp)

    mesh = plsc.VectorSubcoreMesh(core_axis_name="c", subcore_axis_name="s")
    weights = pl.kernel(
        functools.partial(_topk_sc_body, ppw, nchunk),
        mesh=mesh,
        out_type=jax.ShapeDtypeStruct((B, _K2, HW), jnp.float32),
        scratch_types=[
            pltpu.VMEM((_K2, ppw), jnp.float32),
            pltpu.SemaphoreType.DMA,
        ],
    )(sims.reshape(B, _K2, HW))

    out = pl.pallas_call(
        _mix_body,
        grid=(B, H // _TH),
        in_specs=[
            pl.BlockSpec(memory_space=pltpu.SMEM),
            pl.BlockSpec(memory_space=pl.ANY),
            pl.BlockSpec((1, _K2, _TH, W), lambda b, t: (b, 0, t, 0)),
        ],
        out_specs=pl.BlockSpec((1, C, _TH, W), lambda b, t: (b, 0, t, 0)),
        out_shape=jax.ShapeDtypeStruct((B, C, H, W), x.dtype),
        scratch_shapes=[
            pltpu.VMEM((2, C, _SLAB, W + 2 * _R), jnp.float32),
            pltpu.VMEM((_KS, C, _SLAB, W), jnp.float32),
            pltpu.SemaphoreType.DMA((2,)),
        ],
    )(g, xp, weights.reshape(B, _K2, H, W))
    return out
